# baseline (speedup 1.0000x reference)
import jax
import jax.numpy as jnp
from jax import lax
from jax.experimental import pallas as pl
from jax.experimental.pallas import tpu as pltpu
from jax.experimental.pallas import tpu_sc as plsc

NUM_CLASSES = 100
ROWS = 16384 * 50
NUM_CORES = 2
NUM_SUBCORES = 16
NW = NUM_CORES * NUM_SUBCORES
ROWS_PER_W = ROWS // NW
CH = 256
N_CHUNKS = ROWS_PER_W // CH
CHW = CH * NUM_CLASSES
LANES = 16


def _onehot_body(idx_hbm, out_hbm, idx_v, buf0, buf1, sem0, sem1):
    wid = lax.axis_index("s") * NUM_CORES + lax.axis_index("c")
    base = wid * ROWS_PER_W

    pltpu.sync_copy(idx_hbm.at[pl.ds(base, ROWS_PER_W)], idx_v)

    lane100 = lax.iota(jnp.int32, LANES) * NUM_CLASSES
    ones16 = jnp.full((LANES,), 1, jnp.int32)
    zeros16 = jnp.zeros((LANES,), jnp.int32)

    def _zero(k, carry):
        buf0[pl.ds(k * LANES, LANES)] = zeros16
        buf1[pl.ds(k * LANES, LANES)] = zeros16
        return carry
    lax.fori_loop(0, CHW // LANES, _zero, 0)

    def _scatter(buf, c, vals):
        for i in range(CH // LANES):
            idxv = idx_v[pl.ds(c * CH + i * LANES, LANES)]
            pos = idxv + (lane100 + i * LANES * NUM_CLASSES)
            plsc.store_scatter(buf, [pos], vals)

    def _out_slice(c):
        return out_hbm.at[pl.ds(base * NUM_CLASSES + c * CHW, CHW)]

    for b, (buf, sem) in enumerate(((buf0, sem0), (buf1, sem1))):
        _scatter(buf, b, ones16)
        pltpu.async_copy(buf, _out_slice(b), sem)

    def _step(k, carry):
        cc = 2 * k
        for b, (buf, sem) in enumerate(((buf0, sem0), (buf1, sem1))):
            c = cc + b
            pltpu.make_async_copy(buf, _out_slice(c), sem).wait()
            _scatter(buf, c - 2, zeros16)
            _scatter(buf, c, ones16)
            pltpu.async_copy(buf, _out_slice(c), sem)
        return carry
    lax.fori_loop(1, N_CHUNKS // 2, _step, 0)

    pltpu.make_async_copy(buf0, _out_slice(N_CHUNKS - 2), sem0).wait()
    pltpu.make_async_copy(buf1, _out_slice(N_CHUNKS - 1), sem1).wait()


@jax.jit
def kernel(atom_type):
    idx_flat = atom_type.reshape(ROWS)
    mesh = plsc.VectorSubcoreMesh(core_axis_name="c", subcore_axis_name="s")
    out = pl.kernel(
        _onehot_body,
        out_type=jax.ShapeDtypeStruct((ROWS * NUM_CLASSES,), jnp.int32),
        mesh=mesh,
        compiler_params=pltpu.CompilerParams(needs_layout_passes=False),
        scratch_types=[
            pltpu.VMEM((ROWS_PER_W,), jnp.int32),
            pltpu.VMEM((CHW,), jnp.int32),
            pltpu.VMEM((CHW,), jnp.int32),
            pltpu.SemaphoreType.DMA,
            pltpu.SemaphoreType.DMA,
        ],
    )(idx_flat)
    out = out.reshape(atom_type.shape[0], atom_type.shape[1], NUM_CLASSES)
    return jnp.maximum(out, 0)


# speedup vs baseline: 1.8183x; 1.8183x over previous
import jax
import jax.numpy as jnp
from jax import lax
from jax.experimental import pallas as pl
from jax.experimental.pallas import tpu as pltpu
from jax.experimental.pallas import tpu_sc as plsc

NUM_CLASSES = 100
N_ATOMS = 16384
N_PER_ATOM = 50
ROWS = N_ATOMS * N_PER_ATOM
NUM_CORES = 2
NUM_SUBCORES = 16
NW = NUM_CORES * NUM_SUBCORES
ATOMS_PER_W = N_ATOMS // NW
IDX_PER_W = ATOMS_PER_W * N_PER_ATOM
IDX_ALLOC = IDX_PER_W + 16
PR = 56
PC = 128
BR = 50
BC = 100
NBUF = 4
LANES = 16


def _onehot_body(idx_hbm, out_hbm, idx_v, buf0, buf1, buf2, buf3,
                 sem0, sem1, sem2, sem3):
    bufs = (buf0, buf1, buf2, buf3)
    sems = (sem0, sem1, sem2, sem3)
    wid = lax.axis_index("s") * NUM_CORES + lax.axis_index("c")
    atom0 = wid * ATOMS_PER_W

    pltpu.sync_copy(idx_hbm.at[pl.ds(atom0, ATOMS_PER_W)], idx_v)

    lane = lax.iota(jnp.int32, LANES)
    ones16 = jnp.full((LANES,), 1, jnp.int32)
    zeros16 = jnp.zeros((LANES,), jnp.int32)

    def _zero(r, carry):
        rv = jnp.full((LANES,), r, jnp.int32)
        for buf in bufs:
            for o in (0, 16, 32, 48, 64, 80, 84):
                plsc.store_scatter(buf, [rv, lane + o], zeros16)
        return carry
    lax.fori_loop(0, BR, _zero, 0)

    tail_mask = lane < N_PER_ATOM - 3 * LANES

    def _scatter(buf, la, vals):
        lav = jnp.full((LANES,), la, jnp.int32)
        for i in range(4):
            row = lane + i * LANES
            if i == 3:
                row = jnp.minimum(row, BR - 1)
                idxv = plsc.load_gather(idx_v, [lav, row])
                idxv = jnp.minimum(lax.bitwise_and(idxv, PC - 1), BC - 1)
                plsc.store_scatter(buf, [row, idxv], vals, mask=tail_mask)
            else:
                idxv = plsc.load_gather(idx_v, [lav, row])
                plsc.store_scatter(buf, [row, idxv], vals)

    for b in range(NBUF):
        _scatter(bufs[b], b, ones16)
        pltpu.async_copy(bufs[b], out_hbm.at[atom0 + b], sems[b])

    def _step(k, carry):
        aa = NBUF * k
        for b in range(NBUF):
            a = aa + b
            pltpu.make_async_copy(bufs[b], out_hbm.at[atom0 + a],
                                  sems[b]).wait()
            _scatter(bufs[b], a - NBUF, zeros16)
            _scatter(bufs[b], a, ones16)
            pltpu.async_copy(bufs[b], out_hbm.at[atom0 + a], sems[b])
        return carry
    lax.fori_loop(1, ATOMS_PER_W // NBUF, _step, 0)

    for b in range(NBUF):
        pltpu.make_async_copy(
            bufs[b], out_hbm.at[atom0 + ATOMS_PER_W - NBUF + b],
            sems[b]).wait()


@jax.jit
def kernel(atom_type):
    mesh = plsc.VectorSubcoreMesh(core_axis_name="c", subcore_axis_name="s")
    out = pl.kernel(
        _onehot_body,
        out_type=jax.ShapeDtypeStruct((N_ATOMS, N_PER_ATOM, NUM_CLASSES),
                                      jnp.int32),
        mesh=mesh,
        compiler_params=pltpu.CompilerParams(needs_layout_passes=False),
        scratch_types=[
            pltpu.VMEM((ATOMS_PER_W, N_PER_ATOM), jnp.int32),
            pltpu.VMEM((BR, BC), jnp.int32),
            pltpu.VMEM((BR, BC), jnp.int32),
            pltpu.VMEM((BR, BC), jnp.int32),
            pltpu.VMEM((BR, BC), jnp.int32),
            pltpu.SemaphoreType.DMA,
            pltpu.SemaphoreType.DMA,
            pltpu.SemaphoreType.DMA,
            pltpu.SemaphoreType.DMA,
        ],
    )(atom_type)
    return out
